# Initial kernel scaffold; baseline (speedup 1.0000x reference)
import jax
import jax.numpy as jnp
from jax.experimental import pallas as pl


def kernel(atom, edge_index, W, b):
    raise NotImplementedError("write your pallas kernel here")


# speedup vs baseline: 79.0158x; 79.0158x over previous
import functools

import jax
import jax.numpy as jnp
from jax import lax
from jax.experimental import pallas as pl
from jax.experimental.pallas import tpu as pltpu
from jax.experimental.pallas import tpu_sc as plsc


def kernel(atom, edge_index, W, b):
    f32 = jnp.float32
    N, D_IN = atom.shape
    D_OUT = W.shape[0]
    E = edge_index.shape[1]

    NC, NS = 2, 16
    NW = NC * NS
    CHF = 512
    CHD = 2048
    NCH = -(-E // (NW * CHF))
    T = NCH * CHF
    assert T % CHD == 0
    NCHD = T // CHD
    E_pad = NW * T

    NT = 128 * (-(-(N + 1) // (NS * 128)))
    NP = NS * NT
    ZR = 128
    assert NT % ZR == 0 and ZR % 8 == 0

    row = edge_index[0]
    col = edge_index[1]
    pad = E_pad - E
    dummy = jnp.full((pad,), N, jnp.int32)
    row1 = jnp.concatenate([row, dummy])
    col1 = jnp.concatenate([col, dummy])

    mesh = plsc.VectorSubcoreMesh(core_axis_name="c", subcore_axis_name="s",
                                  num_cores=NC, num_subcores=NS)
    sc_params = pltpu.CompilerParams(use_tc_tiling_on_sc=False)

    @functools.partial(
        pl.kernel,
        mesh=mesh,
        out_type=jax.ShapeDtypeStruct((NC * NP,), f32),
        compiler_params=sc_params,
        scratch_types=[
            pltpu.VMEM((CHD,), jnp.int32),
            pltpu.VMEM((CHD,), f32),
            pltpu.VMEM((NT,), f32),
            pltpu.VMEM_SHARED((NP,), f32),
        ],
    )
    def deg_kernel(col_hbm, deg_hbm, colv, onesv, zv, degs):
        c = lax.axis_index("c")
        s = lax.axis_index("s")
        wid = c * NS + s
        ones16 = jnp.full((16,), 1.0, f32)
        zero16 = jnp.zeros((16,), f32)

        def fill_ones(q, carry):
            onesv[pl.ds(q * 16, 16)] = ones16
            return carry

        lax.fori_loop(0, CHD // 16, fill_ones, 0)

        def fill_zero(q, carry):
            zv[pl.ds(q * 16, 16)] = zero16
            return carry

        lax.fori_loop(0, NT // 16, fill_zero, 0)
        pltpu.sync_copy(zv, degs.at[pl.ds(s * NT, NT)])
        plsc.subcore_barrier()

        base = wid * T

        def chunk(ci, carry):
            pltpu.sync_copy(col_hbm.at[pl.ds(base + ci * CHD, CHD)], colv)
            pltpu.sync_copy(onesv, degs.at[colv], add=True)
            return carry

        lax.fori_loop(0, NCHD, chunk, 0)
        plsc.subcore_barrier()
        pltpu.sync_copy(degs.at[pl.ds(s * NT, NT)], zv)
        pltpu.sync_copy(zv, deg_hbm.at[pl.ds(c * NP + s * NT, NT)])

    degp = deg_kernel(col1)

    GB = 16
    NTB = NP // GB

    def lin_body(atom_ref, wt_ref, b_ref, degp_ref, y_ref, dinv_ref):
        x = jnp.dot(atom_ref[...], wt_ref[...], preferred_element_type=f32)
        x = x + b_ref[...]
        dp = degp_ref[...]
        deg = 1.0 + dp[0] + dp[1]
        dinv = lax.rsqrt(deg)
        dinv_ref[...] = dinv
        y_ref[...] = x * dinv

    y, dinv = pl.pallas_call(
        lin_body,
        grid=(GB,),
        in_specs=[
            pl.BlockSpec((NTB, D_IN), lambda i: (i, 0)),
            pl.BlockSpec((D_IN, D_OUT), lambda i: (0, 0)),
            pl.BlockSpec((1, D_OUT), lambda i: (0, 0)),
            pl.BlockSpec((NC, NTB, 1), lambda i: (0, i, 0)),
        ],
        out_specs=[
            pl.BlockSpec((NTB, D_OUT), lambda i: (i, 0)),
            pl.BlockSpec((NTB, 1), lambda i: (i, 0)),
        ],
        out_shape=[
            jax.ShapeDtypeStruct((NP, D_OUT), f32),
            jax.ShapeDtypeStruct((NP, 1), f32),
        ],
    )(atom, W.T, b.reshape(1, D_OUT), degp.reshape(NC, NP, 1))

    @functools.partial(
        pl.kernel,
        mesh=mesh,
        out_type=jax.ShapeDtypeStruct((NC * NP, D_OUT), f32),
        compiler_params=sc_params,
        scratch_types=[
            pltpu.VMEM((CHF,), jnp.int32),
            pltpu.VMEM((CHF,), jnp.int32),
            pltpu.VMEM((CHF,), jnp.int32),
            pltpu.VMEM((CHF,), jnp.int32),
            pltpu.VMEM((CHF, D_OUT), f32),
            pltpu.VMEM((CHF, D_OUT), f32),
            pltpu.VMEM((ZR, D_OUT), f32),
            pltpu.VMEM_SHARED((NP, D_OUT), f32),
            pltpu.SemaphoreType.DMA,
            pltpu.SemaphoreType.DMA,
        ],
    )
    def scat_kernel(y_hbm, row_hbm, col_hbm, acc_hbm,
                    rowa, cola, rowb, colb, msga, msgb, zv, accs,
                    sema, semb):
        c = lax.axis_index("c")
        s = lax.axis_index("s")
        wid = c * NS + s
        zero16 = jnp.zeros((D_OUT,), f32)

        def fz(q, carry):
            zv[q, :] = zero16
            return carry

        lax.fori_loop(0, ZR, fz, 0)

        def zc(k, carry):
            pltpu.sync_copy(zv, accs.at[pl.ds(s * NT + k * ZR, ZR)])
            return carry

        lax.fori_loop(0, NT // ZR, zc, 0)
        plsc.subcore_barrier()

        base = wid * T

        def load_idx(ci, rv, cv):
            pltpu.sync_copy(row_hbm.at[pl.ds(base + ci * CHF, CHF)], rv)
            pltpu.sync_copy(col_hbm.at[pl.ds(base + ci * CHF, CHF)], cv)

        load_idx(0, rowa, cola)
        pltpu.async_copy(y_hbm.at[rowa], msga, sema)

        def body(i, carry):
            c1 = 2 * i + 1
            c2 = 2 * i + 2

            @pl.when(c1 < NCH)
            def _():
                load_idx(c1, rowb, colb)
                pltpu.async_copy(y_hbm.at[rowb], msgb, semb)

            pltpu.make_async_copy(y_hbm.at[rowa], msga, sema).wait()
            pltpu.sync_copy(msga, accs.at[cola], add=True)

            @pl.when(c2 < NCH)
            def _():
                load_idx(c2, rowa, cola)
                pltpu.async_copy(y_hbm.at[rowa], msga, sema)

            @pl.when(c1 < NCH)
            def _():
                pltpu.make_async_copy(y_hbm.at[rowb], msgb, semb).wait()
                pltpu.sync_copy(msgb, accs.at[colb], add=True)

            return carry

        lax.fori_loop(0, (NCH + 1) // 2, body, 0)
        plsc.subcore_barrier()

        def wb(k, carry):
            pltpu.sync_copy(accs.at[pl.ds(s * NT + k * ZR, ZR)], zv)
            pltpu.sync_copy(zv, acc_hbm.at[pl.ds(c * NP + s * NT + k * ZR, ZR)])
            return carry

        lax.fori_loop(0, NT // ZR, wb, 0)

    acc = scat_kernel(y, row1, col1).reshape(NC, NP, D_OUT)

    GD = 100
    ND = N // GD

    def out_body(acc_ref, y_ref, dinv_ref, o_ref):
        a = acc_ref[...]
        t = (a[0] + a[1] + y_ref[...]) * dinv_ref[...]
        o_ref[...] = jnp.maximum(t, 0.0)

    out = pl.pallas_call(
        out_body,
        grid=(GD,),
        in_specs=[
            pl.BlockSpec((NC, ND, D_OUT), lambda i: (0, i, 0)),
            pl.BlockSpec((ND, D_OUT), lambda i: (i, 0)),
            pl.BlockSpec((ND, 1), lambda i: (i, 0)),
        ],
        out_specs=pl.BlockSpec((ND, D_OUT), lambda i: (i, 0)),
        out_shape=jax.ShapeDtypeStruct((N, D_OUT), f32),
    )(acc, y, dinv)

    return out


# speedup vs baseline: 119.4898x; 1.5122x over previous
import functools

import jax
import jax.numpy as jnp
from jax import lax
from jax.experimental import pallas as pl
from jax.experimental.pallas import tpu as pltpu
from jax.experimental.pallas import tpu_sc as plsc


def kernel(atom, edge_index, W, b):
    f32 = jnp.float32
    N, D_IN = atom.shape
    D_OUT = W.shape[0]
    E = edge_index.shape[1]

    NC, NS = 2, 16
    NW = NC * NS
    CHF = 512
    NCH = -(-E // (NW * CHF))
    if NCH % 4:
        NCH += 4 - NCH % 4
    T = NCH * CHF
    E_pad = NW * T

    NT = 128 * (-(-(N + 1) // (NS * 128)))
    NP = NS * NT
    ZR = 128

    CHD = NT
    assert T % CHD == 0 and (T // CHD) % 2 == 0
    NCHD = T // CHD
    assert NT % ZR == 0

    row = edge_index[0]
    col = edge_index[1]
    pad = E_pad - E
    dummy = jnp.full((pad,), N, jnp.int32)
    row1 = jnp.concatenate([row, dummy])
    col1 = jnp.concatenate([col, dummy])

    mesh = plsc.VectorSubcoreMesh(core_axis_name="c", subcore_axis_name="s",
                                  num_cores=NC, num_subcores=NS)
    sc_params = pltpu.CompilerParams(use_tc_tiling_on_sc=False)

    @functools.partial(
        pl.kernel,
        mesh=mesh,
        out_type=jax.ShapeDtypeStruct((NC * NP,), f32),
        compiler_params=sc_params,
        scratch_types=[
            pltpu.VMEM((CHD,), jnp.int32),
            pltpu.VMEM((CHD,), jnp.int32),
            pltpu.VMEM((CHD,), f32),
            pltpu.VMEM((NT,), f32),
            pltpu.VMEM_SHARED((NP,), f32),
            pltpu.SemaphoreType.DMA,
            pltpu.SemaphoreType.DMA,
        ],
    )
    def deg_kernel(col_hbm, deg_hbm, colva, colvb, onesv, zv, degs,
                   sema, semb):
        c = lax.axis_index("c")
        s = lax.axis_index("s")
        wid = c * NS + s
        ones16 = jnp.full((16,), 1.0, f32)
        zero16 = jnp.zeros((16,), f32)

        def fill_ones(q, carry):
            onesv[pl.ds(q * 16, 16)] = ones16
            return carry

        lax.fori_loop(0, CHD // 16, fill_ones, 0)

        def fill_zero(q, carry):
            zv[pl.ds(q * 16, 16)] = zero16
            return carry

        lax.fori_loop(0, NT // 16, fill_zero, 0)
        pltpu.sync_copy(zv, degs.at[pl.ds(s * NT, NT)])
        plsc.subcore_barrier()

        base = wid * T

        def ld(ci, buf, sem):
            return pltpu.async_copy(
                col_hbm.at[pl.ds(base + ci * CHD, CHD)], buf, sem)

        ld(0, colva, sema)

        def chunk2(i, carry):
            c0 = 2 * i
            pltpu.make_async_copy(col_hbm, colva, sema).wait()

            @pl.when(c0 + 1 < NCHD)
            def _():
                ld(c0 + 1, colvb, semb)

            pltpu.sync_copy(onesv, degs.at[colva], add=True)

            @pl.when(c0 + 2 < NCHD)
            def _():
                ld(c0 + 2, colva, sema)

            @pl.when(c0 + 1 < NCHD)
            def _():
                pltpu.make_async_copy(col_hbm, colvb, semb).wait()
                pltpu.sync_copy(onesv, degs.at[colvb], add=True)

            return carry

        lax.fori_loop(0, NCHD // 2, chunk2, 0)
        plsc.subcore_barrier()
        pltpu.sync_copy(degs.at[pl.ds(s * NT, NT)], zv)
        pltpu.sync_copy(zv, deg_hbm.at[pl.ds(c * NP + s * NT, NT)])

    degp = deg_kernel(col1).reshape(NC * NP // 128, 128)

    GN = NP // 2048
    BR = 2048
    DR = BR // 128

    def lin_body(atom_ref, wt_ref, b_ref, dg0_ref, dg1_ref, y_ref):
        x = jnp.dot(atom_ref[...], wt_ref[...], preferred_element_type=f32)
        x = x + b_ref[...]
        deg = 1.0 + dg0_ref[...] + dg1_ref[...]
        dinv_t = lax.transpose(lax.rsqrt(deg), (1, 0))
        for a in range(DR):
            xa = lax.slice(x, (128 * a, 0), (128 * (a + 1), D_OUT))
            da = lax.slice(dinv_t, (0, a), (128, a + 1))
            y_ref[pl.ds(128 * a, 128), :] = xa * da

    y = pl.pallas_call(
        lin_body,
        grid=(GN,),
        in_specs=[
            pl.BlockSpec((BR, D_IN), lambda i: (i, 0)),
            pl.BlockSpec((D_IN, D_OUT), lambda i: (0, 0)),
            pl.BlockSpec((1, D_OUT), lambda i: (0, 0)),
            pl.BlockSpec((DR, 128), lambda i: (i, 0)),
            pl.BlockSpec((DR, 128), lambda i: (GN + i, 0)),
        ],
        out_specs=pl.BlockSpec((BR, D_OUT), lambda i: (i, 0)),
        out_shape=jax.ShapeDtypeStruct((NP, D_OUT), f32),
    )(atom, W.T, b.reshape(1, D_OUT), degp, degp)

    @functools.partial(
        pl.kernel,
        mesh=mesh,
        out_type=jax.ShapeDtypeStruct((NC * NP, D_OUT), f32),
        compiler_params=sc_params,
        scratch_types=[
            [pltpu.VMEM((CHF,), jnp.int32) for _ in range(4)],
            [pltpu.VMEM((CHF,), jnp.int32) for _ in range(4)],
            pltpu.VMEM((CHF, D_OUT), f32),
            pltpu.VMEM((CHF, D_OUT), f32),
            pltpu.VMEM((ZR, D_OUT), f32),
            pltpu.VMEM_SHARED((NP, D_OUT), f32),
            [pltpu.SemaphoreType.DMA for _ in range(4)],
            pltpu.SemaphoreType.DMA,
            pltpu.SemaphoreType.DMA,
            pltpu.SemaphoreType.DMA,
            pltpu.SemaphoreType.DMA,
        ],
    )
    def scat_kernel(y_hbm, row_hbm, col_hbm, acc_hbm,
                    rows, cols, msga, msgb, zv, accs,
                    isems, gsa, gsb, ssa, ssb):
        c = lax.axis_index("c")
        s = lax.axis_index("s")
        wid = c * NS + s
        zero16 = jnp.zeros((D_OUT,), f32)

        def fz(q, carry):
            zv[q, :] = zero16
            return carry

        lax.fori_loop(0, ZR, fz, 0)

        def zc(k, carry):
            pltpu.sync_copy(zv, accs.at[pl.ds(s * NT + k * ZR, ZR)])
            return carry

        lax.fori_loop(0, NT // ZR, zc, 0)
        plsc.subcore_barrier()

        base = wid * T
        msg_bufs = ((msga, gsa, ssa), (msgb, gsb, ssb))

        def start_idx(gi, x):
            pltpu.async_copy(row_hbm.at[pl.ds(base + gi * CHF, CHF)],
                             rows[x], isems[x])
            pltpu.async_copy(col_hbm.at[pl.ds(base + gi * CHF, CHF)],
                             cols[x], isems[x])

        def wait_idx(x):
            pltpu.make_async_copy(row_hbm, rows[x], isems[x]).wait()
            pltpu.make_async_copy(row_hbm, cols[x], isems[x]).wait()

        def start_gather(x, q):
            msg, gs, _ = msg_bufs[q]
            pltpu.async_copy(y_hbm.at[rows[x]], msg, gs)

        def wait_gather(q):
            msg, gs, _ = msg_bufs[q]
            pltpu.make_async_copy(y_hbm, msg, gs).wait()

        def start_scatter(x, q):
            msg, _, ss = msg_bufs[q]
            pltpu.async_copy(msg, accs.at[cols[x]], ss, add=True)

        def wait_scatter(x, q):
            msg, _, ss = msg_bufs[q]
            pltpu.make_async_copy(msg, accs.at[cols[x]], ss).wait()

        for g in range(4):
            start_idx(g, g)
        wait_idx(0)
        start_gather(0, 0)

        def quad(j, carry):
            for k in range(4):
                q = k % 2
                nq = (k + 1) % 2
                xp = (k + 3) % 4

                if k == 0:
                    @pl.when(j > 0)
                    def _():
                        wait_scatter(xp, nq)
                        start_idx(4 * j + k + 3, xp)
                else:
                    wait_scatter(xp, nq)

                    @pl.when(4 * j + k + 3 < NCH)
                    def _():
                        start_idx(4 * j + k + 3, xp)

                if k == 3:
                    @pl.when(j + 1 < NCH // 4)
                    def _():
                        wait_idx(0)
                        start_gather(0, nq)
                else:
                    wait_idx(k + 1)
                    start_gather(k + 1, nq)

                wait_gather(q)
                start_scatter(k, q)
            return carry

        lax.fori_loop(0, NCH // 4, quad, 0)
        wait_scatter(3, 1)
        plsc.subcore_barrier()

        def wb(k, carry):
            pltpu.sync_copy(accs.at[pl.ds(s * NT + k * ZR, ZR)], zv)
            pltpu.sync_copy(zv, acc_hbm.at[pl.ds(c * NP + s * NT + k * ZR, ZR)])
            return carry

        lax.fori_loop(0, NT // ZR, wb, 0)

    acc = scat_kernel(y, row1, col1)

    NPB = NP // BR

    def out_body(a0_ref, a1_ref, y_ref, dg0_ref, dg1_ref, o_ref):
        deg = 1.0 + dg0_ref[...] + dg1_ref[...]
        dinv_t = lax.transpose(lax.rsqrt(deg), (1, 0))
        t = a0_ref[...] + a1_ref[...] + y_ref[...]
        for a in range(DR):
            ta = lax.slice(t, (128 * a, 0), (128 * (a + 1), D_OUT))
            da = lax.slice(dinv_t, (0, a), (128, a + 1))
            o_ref[pl.ds(128 * a, 128), :] = jnp.maximum(ta * da, 0.0)

    out = pl.pallas_call(
        out_body,
        grid=(GN,),
        in_specs=[
            pl.BlockSpec((BR, D_OUT), lambda i: (i, 0)),
            pl.BlockSpec((BR, D_OUT), lambda i: (NPB + i, 0)),
            pl.BlockSpec((BR, D_OUT), lambda i: (i, 0)),
            pl.BlockSpec((DR, 128), lambda i: (i, 0)),
            pl.BlockSpec((DR, 128), lambda i: (GN + i, 0)),
        ],
        out_specs=pl.BlockSpec((BR, D_OUT), lambda i: (i, 0)),
        out_shape=jax.ShapeDtypeStruct((N, D_OUT), f32),
    )(acc, acc, y, degp, degp)

    return out


# speedup vs baseline: 125.1331x; 1.0472x over previous
import functools

import jax
import jax.numpy as jnp
from jax import lax
from jax.experimental import pallas as pl
from jax.experimental.pallas import tpu as pltpu
from jax.experimental.pallas import tpu_sc as plsc


def kernel(atom, edge_index, W, b):
    f32 = jnp.float32
    N, D_IN = atom.shape
    D_OUT = W.shape[0]
    E = edge_index.shape[1]

    NC, NS = 2, 16
    NW = NC * NS
    CHF = 512
    NCH = -(-E // (NW * CHF))
    if NCH % 4:
        NCH += 4 - NCH % 4
    T = NCH * CHF
    E_pad = NW * T

    NT = 128 * (-(-(N + 1) // (NS * 128)))
    NP = NS * NT
    ZR = 128

    CHD = NT
    assert T % CHD == 0 and (T // CHD) % 2 == 0
    NCHD = T // CHD
    assert NT % ZR == 0

    row = edge_index[0]
    col = edge_index[1]
    pad = E_pad - E
    dummy = jnp.full((pad,), N, jnp.int32)
    row1 = jnp.concatenate([row, dummy])
    col1 = jnp.concatenate([col, dummy])

    mesh = plsc.VectorSubcoreMesh(core_axis_name="c", subcore_axis_name="s",
                                  num_cores=NC, num_subcores=NS)
    sc_params = pltpu.CompilerParams(use_tc_tiling_on_sc=False)

    @functools.partial(
        pl.kernel,
        mesh=mesh,
        out_type=jax.ShapeDtypeStruct((NC * NP,), f32),
        compiler_params=sc_params,
        scratch_types=[
            pltpu.VMEM((CHD,), jnp.int32),
            pltpu.VMEM((CHD,), jnp.int32),
            pltpu.VMEM((CHD,), f32),
            pltpu.VMEM((NT,), f32),
            pltpu.VMEM_SHARED((NP,), f32),
            pltpu.SemaphoreType.DMA,
            pltpu.SemaphoreType.DMA,
        ],
    )
    def deg_kernel(col_hbm, deg_hbm, colva, colvb, onesv, zv, degs,
                   sema, semb):
        c = lax.axis_index("c")
        s = lax.axis_index("s")
        wid = c * NS + s
        ones16 = jnp.full((16,), 1.0, f32)
        zero16 = jnp.zeros((16,), f32)

        def fill_ones(q, carry):
            onesv[pl.ds(q * 16, 16)] = ones16
            return carry

        lax.fori_loop(0, CHD // 16, fill_ones, 0)

        def fill_zero(q, carry):
            zv[pl.ds(q * 16, 16)] = zero16
            return carry

        lax.fori_loop(0, NT // 16, fill_zero, 0)
        pltpu.sync_copy(zv, degs.at[pl.ds(s * NT, NT)])
        plsc.subcore_barrier()

        base = wid * T

        def ld(ci, buf, sem):
            return pltpu.async_copy(
                col_hbm.at[pl.ds(base + ci * CHD, CHD)], buf, sem)

        ld(0, colva, sema)

        def chunk2(i, carry):
            c0 = 2 * i
            pltpu.make_async_copy(col_hbm, colva, sema).wait()

            @pl.when(c0 + 1 < NCHD)
            def _():
                ld(c0 + 1, colvb, semb)

            pltpu.sync_copy(onesv, degs.at[colva], add=True)

            @pl.when(c0 + 2 < NCHD)
            def _():
                ld(c0 + 2, colva, sema)

            @pl.when(c0 + 1 < NCHD)
            def _():
                pltpu.make_async_copy(col_hbm, colvb, semb).wait()
                pltpu.sync_copy(onesv, degs.at[colvb], add=True)

            return carry

        lax.fori_loop(0, NCHD // 2, chunk2, 0)
        plsc.subcore_barrier()
        pltpu.sync_copy(degs.at[pl.ds(s * NT, NT)], zv)
        pltpu.sync_copy(zv, deg_hbm.at[pl.ds(c * NP + s * NT, NT)])

    degp = deg_kernel(col1).reshape(NC * NP // 128, 128)

    GN = NP // 2048
    BR = 2048
    DR = BR // 128

    def lin_body(atom_ref, wt_ref, b_ref, dg0_ref, dg1_ref, y_ref, dinv_ref):
        x = jnp.dot(atom_ref[...], wt_ref[...], preferred_element_type=f32)
        x = x + b_ref[...]
        deg = 1.0 + dg0_ref[...] + dg1_ref[...]
        dinv = lax.rsqrt(deg)
        dinv_ref[...] = dinv
        dinv_t = lax.transpose(dinv, (1, 0))
        for a in range(DR):
            xa = lax.slice(x, (128 * a, 0), (128 * (a + 1), D_OUT))
            da = lax.slice(dinv_t, (0, a), (128, a + 1))
            y_ref[pl.ds(128 * a, 128), :] = xa * da

    y, dinvp = pl.pallas_call(
        lin_body,
        grid=(GN,),
        in_specs=[
            pl.BlockSpec((BR, D_IN), lambda i: (i, 0)),
            pl.BlockSpec((D_IN, D_OUT), lambda i: (0, 0)),
            pl.BlockSpec((1, D_OUT), lambda i: (0, 0)),
            pl.BlockSpec((DR, 128), lambda i: (i, 0)),
            pl.BlockSpec((DR, 128), lambda i: (GN + i, 0)),
        ],
        out_specs=[
            pl.BlockSpec((BR, D_OUT), lambda i: (i, 0)),
            pl.BlockSpec((DR, 128), lambda i: (i, 0)),
        ],
        out_shape=[
            jax.ShapeDtypeStruct((NP, D_OUT), f32),
            jax.ShapeDtypeStruct((NP // 128, 128), f32),
        ],
    )(atom, W.T, b.reshape(1, D_OUT), degp, degp)
    dinv1 = dinvp.reshape(NP)

    @functools.partial(
        pl.kernel,
        mesh=mesh,
        out_type=jax.ShapeDtypeStruct((NC * NP, D_OUT), f32),
        compiler_params=sc_params,
        scratch_types=[
            [pltpu.VMEM((CHF,), jnp.int32) for _ in range(4)],
            [pltpu.VMEM((CHF,), jnp.int32) for _ in range(4)],
            pltpu.VMEM((CHF, D_OUT), f32),
            pltpu.VMEM((CHF, D_OUT), f32),
            pltpu.VMEM((ZR, D_OUT), f32),
            pltpu.VMEM_SHARED((NP, D_OUT), f32),
            [pltpu.SemaphoreType.DMA for _ in range(4)],
            pltpu.SemaphoreType.DMA,
            pltpu.SemaphoreType.DMA,
            pltpu.SemaphoreType.DMA,
            pltpu.SemaphoreType.DMA,
        ],
    )
    def scat_kernel(y_hbm, row_hbm, col_hbm, acc_hbm,
                    rows, cols, msga, msgb, zv, accs,
                    isems, gsa, gsb, ssa, ssb):
        c = lax.axis_index("c")
        s = lax.axis_index("s")
        wid = c * NS + s
        zero16 = jnp.zeros((D_OUT,), f32)

        def fz(q, carry):
            zv[q, :] = zero16
            return carry

        lax.fori_loop(0, ZR, fz, 0)

        def zc(k, carry):
            pltpu.sync_copy(zv, accs.at[pl.ds(s * NT + k * ZR, ZR)])
            return carry

        lax.fori_loop(0, NT // ZR, zc, 0)
        plsc.subcore_barrier()

        base = wid * T
        msg_bufs = ((msga, gsa, ssa), (msgb, gsb, ssb))

        def start_idx(gi, x):
            pltpu.async_copy(row_hbm.at[pl.ds(base + gi * CHF, CHF)],
                             rows[x], isems[x])
            pltpu.async_copy(col_hbm.at[pl.ds(base + gi * CHF, CHF)],
                             cols[x], isems[x])

        def wait_idx(x):
            pltpu.make_async_copy(row_hbm, rows[x], isems[x]).wait()
            pltpu.make_async_copy(row_hbm, cols[x], isems[x]).wait()

        def start_gather(x, q):
            msg, gs, _ = msg_bufs[q]
            pltpu.async_copy(y_hbm.at[rows[x]], msg, gs)

        def wait_gather(q):
            msg, gs, _ = msg_bufs[q]
            pltpu.make_async_copy(y_hbm, msg, gs).wait()

        def start_scatter(x, q):
            msg, _, ss = msg_bufs[q]
            pltpu.async_copy(msg, accs.at[cols[x]], ss, add=True)

        def wait_scatter(x, q):
            msg, _, ss = msg_bufs[q]
            pltpu.make_async_copy(msg, accs.at[cols[x]], ss).wait()

        for g in range(4):
            start_idx(g, g)
        wait_idx(0)
        start_gather(0, 0)

        def quad(j, carry):
            for k in range(4):
                q = k % 2
                nq = (k + 1) % 2
                xp = (k + 3) % 4

                if k == 0:
                    @pl.when(j > 0)
                    def _():
                        wait_scatter(xp, nq)
                        start_idx(4 * j + k + 3, xp)
                else:
                    wait_scatter(xp, nq)

                    @pl.when(4 * j + k + 3 < NCH)
                    def _():
                        start_idx(4 * j + k + 3, xp)

                if k == 3:
                    @pl.when(j + 1 < NCH // 4)
                    def _():
                        wait_idx(0)
                        start_gather(0, nq)
                else:
                    wait_idx(k + 1)
                    start_gather(k + 1, nq)

                wait_gather(q)
                start_scatter(k, q)
            return carry

        lax.fori_loop(0, NCH // 4, quad, 0)
        wait_scatter(3, 1)
        plsc.subcore_barrier()

        def wb(k, carry):
            pltpu.sync_copy(accs.at[pl.ds(s * NT + k * ZR, ZR)], zv)
            pltpu.sync_copy(zv, acc_hbm.at[pl.ds(c * NP + s * NT + k * ZR, ZR)])
            return carry

        lax.fori_loop(0, NT // ZR, wb, 0)

    acc = scat_kernel(y, row1, col1)

    NT2 = NP // NW
    CH2 = 448
    NCH2 = NT2 // CH2
    assert NT2 % CH2 == 0 and CH2 % 16 == 0

    @functools.partial(
        pl.kernel,
        mesh=mesh,
        out_type=jax.ShapeDtypeStruct((NP, D_OUT), f32),
        compiler_params=pltpu.CompilerParams(use_tc_tiling_on_sc=False,
                                             needs_layout_passes=False),
        scratch_types=[
            [pltpu.VMEM((CH2, D_OUT), f32) for _ in range(2)],
            [pltpu.VMEM((CH2, D_OUT), f32) for _ in range(2)],
            [pltpu.VMEM((CH2, D_OUT), f32) for _ in range(2)],
            [pltpu.VMEM((CH2,), f32) for _ in range(2)],
            [pltpu.VMEM((CH2, D_OUT), f32) for _ in range(2)],
            [pltpu.SemaphoreType.DMA for _ in range(2)],
            [pltpu.SemaphoreType.DMA for _ in range(2)],
        ],
    )
    def comb_kernel(acc_hbm, y_hbm, dinv_hbm, out_hbm,
                    a0b, a1b, yb, dvb, ob, isem, osem):
        c = lax.axis_index("c")
        s = lax.axis_index("s")
        base = (c * NS + s) * NT2

        def start_in(k, p):
            off = base + k * CH2
            pltpu.async_copy(acc_hbm.at[pl.ds(off, CH2)], a0b[p], isem[p])
            pltpu.async_copy(acc_hbm.at[pl.ds(NP + off, CH2)], a1b[p], isem[p])
            pltpu.async_copy(y_hbm.at[pl.ds(off, CH2)], yb[p], isem[p])
            pltpu.async_copy(dinv_hbm.at[pl.ds(off, CH2)], dvb[p], isem[p])

        def wait_in(p):
            pltpu.make_async_copy(acc_hbm, a0b[p], isem[p]).wait()
            pltpu.make_async_copy(acc_hbm, a1b[p], isem[p]).wait()
            pltpu.make_async_copy(y_hbm, yb[p], isem[p]).wait()
            pltpu.make_async_copy(dinv_hbm, dvb[p], isem[p]).wait()

        start_in(0, 0)
        for k in range(NCH2):
            p = k % 2
            if k + 1 < NCH2:
                start_in(k + 1, (k + 1) % 2)
            wait_in(p)
            if k >= 2:
                pltpu.make_async_copy(
                    ob[p], out_hbm.at[pl.ds(base + (k - 2) * CH2, CH2)],
                    osem[p]).wait()

            def group(g, carry):
                for u in range(16):
                    n = g * 16 + u
                    dv = plsc.load_gather(
                        dvb[p], [jnp.full((16,), n, jnp.int32)])
                    t = a0b[p][n, :] + a1b[p][n, :] + yb[p][n, :]
                    ob[p][n, :] = jnp.maximum(t * dv, 0.0)
                return carry

            lax.fori_loop(0, CH2 // 16, group, 0)
            pltpu.async_copy(ob[p], out_hbm.at[pl.ds(base + k * CH2, CH2)],
                             osem[p])
        for k in (NCH2 - 2, NCH2 - 1):
            pltpu.make_async_copy(
                ob[k % 2], out_hbm.at[pl.ds(base + k * CH2, CH2)],
                osem[k % 2]).wait()

    outp = comb_kernel(acc, y, dinv1)
    return outp[:N]


# speedup vs baseline: 128.9690x; 1.0307x over previous
import functools

import jax
import jax.numpy as jnp
from jax import lax
from jax.experimental import pallas as pl
from jax.experimental.pallas import tpu as pltpu
from jax.experimental.pallas import tpu_sc as plsc


def kernel(atom, edge_index, W, b):
    f32 = jnp.float32
    N, D_IN = atom.shape
    D_OUT = W.shape[0]
    E = edge_index.shape[1]

    NC, NS = 2, 16
    NW = NC * NS
    CHF = 512
    NCH = -(-E // (NW * CHF))
    if NCH % 4:
        NCH += 4 - NCH % 4
    T = NCH * CHF
    E_pad = NW * T

    NT = 128 * (-(-(N + 1) // (NS * 128)))
    NP = NS * NT
    ZR = 128

    CHD = NT
    assert T % CHD == 0 and (T // CHD) % 2 == 0
    NCHD = T // CHD
    assert NT % ZR == 0

    row = edge_index[0]
    col = edge_index[1]
    pad = E_pad - E
    dummy = jnp.full((pad,), N, jnp.int32)
    row1 = jnp.concatenate([row, dummy])
    col1 = jnp.concatenate([col, dummy])

    mesh = plsc.VectorSubcoreMesh(core_axis_name="c", subcore_axis_name="s",
                                  num_cores=NC, num_subcores=NS)
    sc_params = pltpu.CompilerParams(use_tc_tiling_on_sc=False)

    @functools.partial(
        pl.kernel,
        mesh=mesh,
        out_type=jax.ShapeDtypeStruct((NC * NP,), f32),
        compiler_params=sc_params,
        scratch_types=[
            pltpu.VMEM((CHD,), jnp.int32),
            pltpu.VMEM((CHD,), jnp.int32),
            pltpu.VMEM((CHD,), f32),
            pltpu.VMEM((NT,), f32),
            pltpu.VMEM_SHARED((NP,), f32),
            pltpu.SemaphoreType.DMA,
            pltpu.SemaphoreType.DMA,
        ],
    )
    def deg_kernel(col_hbm, deg_hbm, colva, colvb, onesv, zv, degs,
                   sema, semb):
        c = lax.axis_index("c")
        s = lax.axis_index("s")
        wid = c * NS + s
        ones16 = jnp.full((16,), 1.0, f32)
        zero16 = jnp.zeros((16,), f32)

        def fill_ones(q, carry):
            onesv[pl.ds(q * 16, 16)] = ones16
            return carry

        lax.fori_loop(0, CHD // 16, fill_ones, 0)

        def fill_zero(q, carry):
            zv[pl.ds(q * 16, 16)] = zero16
            return carry

        lax.fori_loop(0, NT // 16, fill_zero, 0)
        pltpu.sync_copy(zv, degs.at[pl.ds(s * NT, NT)])
        plsc.subcore_barrier()

        base = wid * T

        def ld(ci, buf, sem):
            return pltpu.async_copy(
                col_hbm.at[pl.ds(base + ci * CHD, CHD)], buf, sem)

        ld(0, colva, sema)

        def chunk2(i, carry):
            c0 = 2 * i
            pltpu.make_async_copy(col_hbm, colva, sema).wait()

            @pl.when(c0 + 1 < NCHD)
            def _():
                ld(c0 + 1, colvb, semb)

            pltpu.sync_copy(onesv, degs.at[colva], add=True)

            @pl.when(c0 + 2 < NCHD)
            def _():
                ld(c0 + 2, colva, sema)

            @pl.when(c0 + 1 < NCHD)
            def _():
                pltpu.make_async_copy(col_hbm, colvb, semb).wait()
                pltpu.sync_copy(onesv, degs.at[colvb], add=True)

            return carry

        lax.fori_loop(0, NCHD // 2, chunk2, 0)
        plsc.subcore_barrier()
        pltpu.sync_copy(degs.at[pl.ds(s * NT, NT)], zv)
        pltpu.sync_copy(zv, deg_hbm.at[pl.ds(c * NP + s * NT, NT)])

    degp = deg_kernel(col1).reshape(NC * NP // 128, 128)

    GN = NP // 2048
    BR = 2048
    DR = BR // 128

    def lin_body(atom_ref, wt_ref, b_ref, dg0_ref, dg1_ref, y_ref, dinv_ref):
        x = jnp.dot(atom_ref[...], wt_ref[...], preferred_element_type=f32)
        x = x + b_ref[...]
        deg = 1.0 + dg0_ref[...] + dg1_ref[...]
        dinv = lax.rsqrt(deg)
        dinv_ref[...] = dinv
        dinv_t = lax.transpose(dinv, (1, 0))
        pieces = []
        for a in range(DR):
            xa = lax.slice(x, (128 * a, 0), (128 * (a + 1), D_OUT))
            da = lax.slice(dinv_t, (0, a), (128, a + 1))
            pieces.append(xa * da)
        yv = lax.concatenate(pieces, 0)
        y3 = yv.reshape(BR // 8, 8, D_OUT)
        packed = lax.concatenate(
            [lax.squeeze(lax.slice(y3, (0, u, 0), (BR // 8, u + 1, D_OUT)),
                         (1,)) for u in range(8)], 1)
        y_ref[...] = packed

    yp, dinvp = pl.pallas_call(
        lin_body,
        grid=(GN,),
        in_specs=[
            pl.BlockSpec((BR, D_IN), lambda i: (i, 0)),
            pl.BlockSpec((D_IN, D_OUT), lambda i: (0, 0)),
            pl.BlockSpec((1, D_OUT), lambda i: (0, 0)),
            pl.BlockSpec((DR, 128), lambda i: (i, 0)),
            pl.BlockSpec((DR, 128), lambda i: (GN + i, 0)),
        ],
        out_specs=[
            pl.BlockSpec((BR // 8, 128), lambda i: (i, 0)),
            pl.BlockSpec((DR, 128), lambda i: (i, 0)),
        ],
        out_shape=[
            jax.ShapeDtypeStruct((NP // 8, 128), f32),
            jax.ShapeDtypeStruct((NP // 128, 128), f32),
        ],
    )(atom, W.T, b.reshape(1, D_OUT), degp, degp)
    y = yp.reshape(NP, D_OUT)

    @functools.partial(
        pl.kernel,
        mesh=mesh,
        out_type=jax.ShapeDtypeStruct((NC * NP, D_OUT), f32),
        compiler_params=sc_params,
        scratch_types=[
            [pltpu.VMEM((CHF,), jnp.int32) for _ in range(4)],
            [pltpu.VMEM((CHF,), jnp.int32) for _ in range(4)],
            pltpu.VMEM((CHF, D_OUT), f32),
            pltpu.VMEM((CHF, D_OUT), f32),
            pltpu.VMEM((ZR, D_OUT), f32),
            pltpu.VMEM_SHARED((NP, D_OUT), f32),
            [pltpu.SemaphoreType.DMA for _ in range(4)],
            pltpu.SemaphoreType.DMA,
            pltpu.SemaphoreType.DMA,
            pltpu.SemaphoreType.DMA,
            pltpu.SemaphoreType.DMA,
        ],
    )
    def scat_kernel(y_hbm, row_hbm, col_hbm, acc_hbm,
                    rows, cols, msga, msgb, zv, accs,
                    isems, gsa, gsb, ssa, ssb):
        c = lax.axis_index("c")
        s = lax.axis_index("s")
        wid = c * NS + s
        zero16 = jnp.zeros((D_OUT,), f32)

        def fz(q, carry):
            zv[q, :] = zero16
            return carry

        lax.fori_loop(0, ZR, fz, 0)

        def zc(k, carry):
            pltpu.sync_copy(zv, accs.at[pl.ds(s * NT + k * ZR, ZR)])
            return carry

        lax.fori_loop(0, NT // ZR, zc, 0)
        plsc.subcore_barrier()

        base = wid * T
        msg_bufs = ((msga, gsa, ssa), (msgb, gsb, ssb))

        def start_idx(gi, x):
            pltpu.async_copy(row_hbm.at[pl.ds(base + gi * CHF, CHF)],
                             rows[x], isems[x])
            pltpu.async_copy(col_hbm.at[pl.ds(base + gi * CHF, CHF)],
                             cols[x], isems[x])

        def wait_idx(x):
            pltpu.make_async_copy(row_hbm, rows[x], isems[x]).wait()
            pltpu.make_async_copy(row_hbm, cols[x], isems[x]).wait()

        def start_gather(x, q):
            msg, gs, _ = msg_bufs[q]
            pltpu.async_copy(y_hbm.at[rows[x]], msg, gs)

        def wait_gather(q):
            msg, gs, _ = msg_bufs[q]
            pltpu.make_async_copy(y_hbm, msg, gs).wait()

        def start_scatter(x, q):
            msg, _, ss = msg_bufs[q]
            pltpu.async_copy(msg, accs.at[cols[x]], ss, add=True)

        def wait_scatter(x, q):
            msg, _, ss = msg_bufs[q]
            pltpu.make_async_copy(msg, accs.at[cols[x]], ss).wait()

        for g in range(4):
            start_idx(g, g)
        wait_idx(0)
        start_gather(0, 0)

        def quad(j, carry):
            for k in range(4):
                q = k % 2
                nq = (k + 1) % 2
                xp = (k + 3) % 4

                if k == 0:
                    @pl.when(j > 0)
                    def _():
                        wait_scatter(xp, nq)
                        start_idx(4 * j + k + 3, xp)
                else:
                    wait_scatter(xp, nq)

                    @pl.when(4 * j + k + 3 < NCH)
                    def _():
                        start_idx(4 * j + k + 3, xp)

                if k == 3:
                    @pl.when(j + 1 < NCH // 4)
                    def _():
                        wait_idx(0)
                        start_gather(0, nq)
                else:
                    wait_idx(k + 1)
                    start_gather(k + 1, nq)

                wait_gather(q)
                start_scatter(k, q)
            return carry

        lax.fori_loop(0, NCH // 4, quad, 0)
        wait_scatter(3, 1)
        plsc.subcore_barrier()

        def wb(k, carry):
            pltpu.sync_copy(accs.at[pl.ds(s * NT + k * ZR, ZR)], zv)
            pltpu.sync_copy(zv, acc_hbm.at[pl.ds(c * NP + s * NT + k * ZR, ZR)])
            return carry

        lax.fori_loop(0, NT // ZR, wb, 0)

    acc = scat_kernel(y, row1, col1)
    accp = acc.reshape(NC * NP // 8, 128)

    PB = BR // 8
    NPB = NP // 8 // PB

    def out_body(a0_ref, a1_ref, y_ref, dg0_ref, dg1_ref, o_ref):
        tp = a0_ref[...] + a1_ref[...] + y_ref[...]
        pieces = [
            lax.reshape(lax.slice(tp, (0, D_OUT * u), (PB, D_OUT * (u + 1))),
                        (PB, 1, D_OUT)) for u in range(8)
        ]
        t = lax.concatenate(pieces, 1).reshape(BR, D_OUT)
        deg = 1.0 + dg0_ref[...] + dg1_ref[...]
        dinv_t = lax.transpose(lax.rsqrt(deg), (1, 0))
        for a in range(DR):
            ta = lax.slice(t, (128 * a, 0), (128 * (a + 1), D_OUT))
            da = lax.slice(dinv_t, (0, a), (128, a + 1))
            o_ref[pl.ds(128 * a, 128), :] = jnp.maximum(ta * da, 0.0)

    out = pl.pallas_call(
        out_body,
        grid=(GN,),
        in_specs=[
            pl.BlockSpec((PB, 128), lambda i: (i, 0)),
            pl.BlockSpec((PB, 128), lambda i: (NPB + i, 0)),
            pl.BlockSpec((PB, 128), lambda i: (i, 0)),
            pl.BlockSpec((DR, 128), lambda i: (i, 0)),
            pl.BlockSpec((DR, 128), lambda i: (GN + i, 0)),
        ],
        out_specs=pl.BlockSpec((BR, D_OUT), lambda i: (i, 0)),
        out_shape=jax.ShapeDtypeStruct((N, D_OUT), f32),
    )(accp, accp, yp, degp, degp)

    return out
